# Initial kernel scaffold; baseline (speedup 1.0000x reference)
import jax
import jax.numpy as jnp
from jax.experimental import pallas as pl


def kernel(nbr_vec, angle_nbr_idx):
    raise NotImplementedError("write your pallas kernel here")


# speedup vs baseline: 140.9548x; 140.9548x over previous
import functools

import jax
import jax.numpy as jnp
from jax import lax
from jax.experimental import pallas as pl
from jax.experimental.pallas import tpu as pltpu
from jax.experimental.pallas import tpu_sc as plsc

A = 6_400_000
NW = 32
CHUNK = 4_000
PER_W = A // NW
N_CHUNKS = PER_W // CHUNK

_C0 = 0.99998007
_C1 = -0.33269442
_C2 = 0.19401986
_C3 = -0.11769517
_C4 = 0.05408272
_C5 = -0.01229974

_HALF_PI = 1.5707963267948966
_PI = 3.141592653589793


def _angle_16(a, b, c, d, e, f):
    x = -(a * d + b * e + c * f)
    cx = b * f - c * e
    cy = c * d - a * f
    cz = a * e - b * d
    s = cx * cx + cy * cy + cz * cz
    i = lax.bitcast_convert_type(s, jnp.int32)
    i = jnp.int32(0x5F3759DF) - lax.shift_right_logical(i, 1)
    r = lax.bitcast_convert_type(i, jnp.float32)
    r = r * (1.5 - 0.5 * s * r * r)
    r = r * (1.5 - 0.5 * s * r * r)
    y = jnp.maximum(s * r, 1e-9)
    ax = jnp.abs(x)
    mn = jnp.minimum(ax, y)
    mx = jnp.maximum(ax, y)
    t = mn / mx
    t2 = t * t
    p = _C5
    p = p * t2 + _C4
    p = p * t2 + _C3
    p = p * t2 + _C2
    p = p * t2 + _C1
    p = p * t2 + _C0
    p = p * t
    base = jnp.where(ax > y, p, _HALF_PI - p)
    return jnp.where(x >= 0, base, _PI - base)


@functools.partial(
    pl.kernel,
    mesh=plsc.VectorSubcoreMesh(core_axis_name="c", subcore_axis_name="s"),
    out_type=jax.ShapeDtypeStruct((A,), jnp.float32),
    scratch_types=[
        pltpu.VMEM((CHUNK,), jnp.int32),
        pltpu.VMEM((CHUNK,), jnp.int32),
        pltpu.VMEM((CHUNK,), jnp.float32),
        pltpu.VMEM((CHUNK,), jnp.float32),
        pltpu.VMEM((CHUNK,), jnp.float32),
        pltpu.VMEM((CHUNK,), jnp.float32),
        pltpu.VMEM((CHUNK,), jnp.float32),
        pltpu.VMEM((CHUNK,), jnp.float32),
        pltpu.VMEM((CHUNK,), jnp.float32),
        pltpu.SemaphoreType.DMA,
    ],
)
def _angle_sc(nx_hbm, ny_hbm, nz_hbm, idx0_hbm, idx1_hbm, out_hbm,
              idx0_v, idx1_v, a_v, b_v, c_v, d_v, e_v, f_v, out_v, sem):
    wid = lax.axis_index("s") * 2 + lax.axis_index("c")
    w_base = wid * PER_W

    def chunk_body(k, carry):
        base = w_base + k * CHUNK
        pltpu.sync_copy(idx0_hbm.at[pl.ds(base, CHUNK)], idx0_v)
        pltpu.sync_copy(idx1_hbm.at[pl.ds(base, CHUNK)], idx1_v)
        cps = [
            pltpu.async_copy(nx_hbm.at[idx0_v], a_v, sem),
            pltpu.async_copy(ny_hbm.at[idx0_v], b_v, sem),
            pltpu.async_copy(nz_hbm.at[idx0_v], c_v, sem),
            pltpu.async_copy(nx_hbm.at[idx1_v], d_v, sem),
            pltpu.async_copy(ny_hbm.at[idx1_v], e_v, sem),
            pltpu.async_copy(nz_hbm.at[idx1_v], f_v, sem),
        ]
        for cp in cps:
            cp.wait()

        def compute(i, carry2):
            sl = pl.ds(i * 16, 16)
            out_v[sl] = _angle_16(a_v[sl], b_v[sl], c_v[sl],
                                  d_v[sl], e_v[sl], f_v[sl])
            return carry2

        lax.fori_loop(0, CHUNK // 16, compute, 0)
        pltpu.sync_copy(out_v, out_hbm.at[pl.ds(base, CHUNK)])
        return carry

    lax.fori_loop(0, N_CHUNKS, chunk_body, 0)


def kernel(nbr_vec, angle_nbr_idx):
    nx = nbr_vec[:, 0]
    ny = nbr_vec[:, 1]
    nz = nbr_vec[:, 2]
    idx0 = angle_nbr_idx[:, 0]
    idx1 = angle_nbr_idx[:, 1]
    return _angle_sc(nx, ny, nz, idx0, idx1)


# speedup vs baseline: 160.4552x; 1.1383x over previous
import functools

import jax
import jax.numpy as jnp
from jax import lax
from jax.experimental import pallas as pl
from jax.experimental.pallas import tpu as pltpu
from jax.experimental.pallas import tpu_sc as plsc

A = 6_400_000
NW = 32
CHUNK = 4_000
PER_W = A // NW
N_CHUNKS = PER_W // CHUNK

_C0 = 0.99998007
_C1 = -0.33269442
_C2 = 0.19401986
_C3 = -0.11769517
_C4 = 0.05408272
_C5 = -0.01229974

_HALF_PI = 1.5707963267948966
_PI = 3.141592653589793


def _angle_16(a, b, c, d, e, f):
    x = -(a * d + b * e + c * f)
    cx = b * f - c * e
    cy = c * d - a * f
    cz = a * e - b * d
    s = cx * cx + cy * cy + cz * cz
    i = lax.bitcast_convert_type(s, jnp.int32)
    i = jnp.int32(0x5F3759DF) - lax.shift_right_logical(i, 1)
    r = lax.bitcast_convert_type(i, jnp.float32)
    r = r * (1.5 - 0.5 * s * r * r)
    r = r * (1.5 - 0.5 * s * r * r)
    y = jnp.maximum(s * r, 1e-9)
    ax = jnp.abs(x)
    mn = jnp.minimum(ax, y)
    mx = jnp.maximum(ax, y)
    t = mn / mx
    t2 = t * t
    p = _C5
    p = p * t2 + _C4
    p = p * t2 + _C3
    p = p * t2 + _C2
    p = p * t2 + _C1
    p = p * t2 + _C0
    p = p * t
    base = jnp.where(ax > y, p, _HALF_PI - p)
    return jnp.where(x >= 0, base, _PI - base)


_COMP_BUF = lambda: pltpu.VMEM((CHUNK,), jnp.float32)
_IDX_BUF = lambda: pltpu.VMEM((CHUNK,), jnp.int32)


@functools.partial(
    pl.kernel,
    mesh=plsc.VectorSubcoreMesh(core_axis_name="c", subcore_axis_name="s"),
    out_type=jax.ShapeDtypeStruct((A,), jnp.float32),
    scratch_types=[
        _IDX_BUF(), _IDX_BUF(),
        _IDX_BUF(), _IDX_BUF(),
        _COMP_BUF(), _COMP_BUF(), _COMP_BUF(),
        _COMP_BUF(), _COMP_BUF(), _COMP_BUF(),
        _COMP_BUF(), _COMP_BUF(), _COMP_BUF(),
        _COMP_BUF(), _COMP_BUF(), _COMP_BUF(),
        pltpu.VMEM((CHUNK,), jnp.float32),
        pltpu.SemaphoreType.DMA,
        pltpu.SemaphoreType.DMA,
    ],
)
def _angle_sc(nx_hbm, ny_hbm, nz_hbm, idx0_hbm, idx1_hbm, out_hbm,
              idx0_v0, idx1_v0, idx0_v1, idx1_v1,
              a0, b0, c0, d0, e0, f0,
              a1, b1, c1, d1, e1, f1,
              out_v, sem0, sem1):
    wid = lax.axis_index("s") * 2 + lax.axis_index("c")
    w_base = wid * PER_W

    idx_bufs = ((idx0_v0, idx1_v0), (idx0_v1, idx1_v1))
    comp_bufs = ((a0, b0, c0, d0, e0, f0), (a1, b1, c1, d1, e1, f1))
    sems = (sem0, sem1)

    def fire(k, slot):
        base = w_base + k * CHUNK
        i0, i1 = idx_bufs[slot]
        a, b, c, d, e, f = comp_bufs[slot]
        sem = sems[slot]
        pltpu.sync_copy(idx0_hbm.at[pl.ds(base, CHUNK)], i0)
        pltpu.sync_copy(idx1_hbm.at[pl.ds(base, CHUNK)], i1)
        pltpu.async_copy(nx_hbm.at[i0], a, sem)
        pltpu.async_copy(ny_hbm.at[i0], b, sem)
        pltpu.async_copy(nz_hbm.at[i0], c, sem)
        pltpu.async_copy(nx_hbm.at[i1], d, sem)
        pltpu.async_copy(ny_hbm.at[i1], e, sem)
        pltpu.async_copy(nz_hbm.at[i1], f, sem)

    def drain(slot):
        i0, _ = idx_bufs[slot]
        a, b, c, d, e, f = comp_bufs[slot]
        sem = sems[slot]
        for dst, src in ((a, nx_hbm), (b, ny_hbm), (c, nz_hbm),
                         (d, nx_hbm), (e, ny_hbm), (f, nz_hbm)):
            pltpu.make_async_copy(src.at[i0], dst, sem).wait()

    def compute_and_store(k, slot):
        a, b, c, d, e, f = comp_bufs[slot]

        def compute(i, carry2):
            sl = pl.ds(i * 16, 16)
            out_v[sl] = _angle_16(a[sl], b[sl], c[sl], d[sl], e[sl], f[sl])
            return carry2

        lax.fori_loop(0, CHUNK // 16, compute, 0, unroll=2)
        base = w_base + k * CHUNK
        pltpu.sync_copy(out_v, out_hbm.at[pl.ds(base, CHUNK)])

    fire(0, 0)

    def chunk_body(k, carry):
        for step in range(2):
            kk = 2 * k + step
            slot = step
            fire(kk + 1, 1 - slot)
            drain(slot)
            compute_and_store(kk, slot)
        return carry

    lax.fori_loop(0, N_CHUNKS // 2 - 1, chunk_body, 0)

    k_last = N_CHUNKS - 2
    fire(k_last + 1, 1)
    drain(0)
    compute_and_store(k_last, 0)
    drain(1)
    compute_and_store(k_last + 1, 1)


def kernel(nbr_vec, angle_nbr_idx):
    nx = nbr_vec[:, 0]
    ny = nbr_vec[:, 1]
    nz = nbr_vec[:, 2]
    idx0 = angle_nbr_idx[:, 0]
    idx1 = angle_nbr_idx[:, 1]
    return _angle_sc(nx, ny, nz, idx0, idx1)
